# Initial kernel scaffold; baseline (speedup 1.0000x reference)
import jax
import jax.numpy as jnp
from jax.experimental import pallas as pl


def kernel(x, edge_index, h_node, W_gcn, b_gcn, W_fus, b_fus, W_cls, b_cls):
    raise NotImplementedError("write your pallas kernel here")


# speedup vs baseline: 11.6250x; 11.6250x over previous
import functools

import jax
import jax.numpy as jnp
from jax import lax
from jax.experimental import pallas as pl
from jax.experimental.pallas import tpu as pltpu
from jax.experimental.pallas import tpu_sc as plsc

NN = 10000
DIN = 256
DHID = 256
HALF = 128
NP = 10240
TROW = NP // 16
EB = 128
NCORE = 2
NSUB = 16

_mesh = plsc.VectorSubcoreMesh(core_axis_name="c", subcore_axis_name="s")


def _deg_sc(epad):
    e_tile = epad // (NCORE * NSUB)
    nb = e_tile // EB

    @functools.partial(
        pl.kernel,
        mesh=_mesh,
        out_type=jax.ShapeDtypeStruct((NCORE, NP), jnp.float32),
        scratch_types=[
            pltpu.VMEM((1, EB), jnp.int32),
            pltpu.VMEM((EB,), jnp.float32),
            pltpu.VMEM((TROW,), jnp.float32),
            pltpu.VMEM_SHARED((NP,), jnp.float32),
        ],
    )
    def k(dst_hbm, zrow_hbm, deg_out, idx_v, ones_v, zbuf_v, deg_sh):
        c = lax.axis_index("c")
        s = lax.axis_index("s")
        wid = s * NCORE + c
        for i in range(EB // 16):
            ones_v[pl.ds(i * 16, 16)] = jnp.ones((16,), jnp.float32)
        pltpu.sync_copy(zrow_hbm, zbuf_v)
        pltpu.sync_copy(zbuf_v, deg_sh.at[pl.ds(s * TROW, TROW)])
        plsc.subcore_barrier()

        def body(b, carry):
            base = wid * e_tile + b * EB
            pltpu.sync_copy(dst_hbm.at[pl.ds(base, EB)], idx_v.at[0])
            pltpu.sync_copy(ones_v, deg_sh.at[idx_v.at[0]], add=True)
            return carry

        lax.fori_loop(0, nb, body, 0)
        plsc.subcore_barrier()
        pltpu.sync_copy(deg_sh.at[pl.ds(s * TROW, TROW)],
                        deg_out.at[c, pl.ds(s * TROW, TROW)])

    return k


def _msg_sc(epad):
    e_tile = epad // NSUB
    nb = e_tile // EB

    @functools.partial(
        pl.kernel,
        mesh=_mesh,
        out_type=jax.ShapeDtypeStruct((NCORE, NP, HALF), jnp.float32),
        scratch_types=[
            pltpu.VMEM((1, EB), jnp.int32),
            pltpu.VMEM((1, EB), jnp.int32),
            pltpu.VMEM((EB, HALF), jnp.float32),
            pltpu.VMEM_SHARED((NP, HALF), jnp.float32),
            pltpu.SemaphoreType.DMA,
        ],
    )
    def k(table_hbm, src2_hbm, dst_hbm, zrows_hbm, acc_out,
          sidx_v, didx_v, rows_v, acc_sh, sem):
        c = lax.axis_index("c")
        s = lax.axis_index("s")
        pltpu.sync_copy(zrows_hbm.at[pl.ds(s * TROW, TROW)],
                        acc_sh.at[pl.ds(s * TROW, TROW)])
        plsc.subcore_barrier()

        def body(b, carry):
            base = s * e_tile + b * EB
            pltpu.sync_copy(src2_hbm.at[c, pl.ds(base, EB)], sidx_v.at[0])
            pltpu.sync_copy(dst_hbm.at[pl.ds(base, EB)], didx_v.at[0])
            pltpu.async_copy(table_hbm.at[sidx_v.at[0]], rows_v, sem).wait()
            pltpu.sync_copy(rows_v, acc_sh.at[didx_v.at[0]], add=True)
            return carry

        lax.fori_loop(0, nb, body, 0)
        plsc.subcore_barrier()
        pltpu.sync_copy(acc_sh.at[pl.ds(s * TROW, TROW)],
                        acc_out.at[c, pl.ds(s * TROW, TROW)])

    return k


_BN = 400
_NBLK = NN // _BN


def _mm_body(x_ref, w_ref, o_ref):
    o_ref[...] = jnp.dot(x_ref[...], w_ref[...],
                         preferred_element_type=jnp.float32)


def _xw_tc(x, w):
    return pl.pallas_call(
        _mm_body,
        grid=(_NBLK,),
        in_specs=[pl.BlockSpec((_BN, DIN), lambda i: (i, 0)),
                  pl.BlockSpec((DIN, DHID), lambda i: (0, 0))],
        out_specs=pl.BlockSpec((_BN, DHID), lambda i: (i, 0)),
        out_shape=jax.ShapeDtypeStruct((NN, DHID), jnp.float32),
    )(x, w)


def _scale_body(xw_ref, deg_ref, tab_ref, dis_ref):
    deg = deg_ref[0] + deg_ref[1] + 1.0
    dis = lax.rsqrt(deg)
    dis_ref[...] = dis
    tab_ref[0] = xw_ref[:, :HALF] * dis
    tab_ref[1] = xw_ref[:, HALF:] * dis


def _scale_tc(xw, deg3):
    return pl.pallas_call(
        _scale_body,
        grid=(_NBLK,),
        in_specs=[pl.BlockSpec((_BN, DHID), lambda i: (i, 0)),
                  pl.BlockSpec((NCORE, _BN, 1), lambda i: (0, i, 0))],
        out_specs=[pl.BlockSpec((NCORE, _BN, HALF), lambda i: (0, i, 0)),
                   pl.BlockSpec((_BN, 1), lambda i: (i, 0))],
        out_shape=[jax.ShapeDtypeStruct((NCORE, NN, HALF), jnp.float32),
                   jax.ShapeDtypeStruct((NN, 1), jnp.float32)],
    )(xw, deg3)


def _epi_body(acc_ref, tab_ref, dis_ref, hn_ref, bg_ref, wf_ref, bf_ref,
              wc_ref, bc_ref, o_ref):
    a = jnp.concatenate(
        [acc_ref[0] + tab_ref[0], acc_ref[1] + tab_ref[1]], axis=1)
    h = jnp.maximum(a * dis_ref[...] + bg_ref[...], 0.0)
    ha = (1.0 - hn_ref[...]) * h
    h2 = jnp.maximum(
        jnp.dot(ha, wf_ref[...], preferred_element_type=jnp.float32)
        + bf_ref[...], 0.0)
    o_ref[...] = (jnp.dot(h2, wc_ref[...], preferred_element_type=jnp.float32)
                  + bc_ref[...])


def _epi_tc(acc, tab, dis, hn, bg, wf, bf, wc, bc):
    nc = wc.shape[1]
    return pl.pallas_call(
        _epi_body,
        grid=(_NBLK,),
        in_specs=[pl.BlockSpec((NCORE, _BN, HALF), lambda i: (0, i, 0)),
                  pl.BlockSpec((NCORE, _BN, HALF), lambda i: (0, i, 0)),
                  pl.BlockSpec((_BN, 1), lambda i: (i, 0)),
                  pl.BlockSpec((_BN, 1), lambda i: (i, 0)),
                  pl.BlockSpec((1, DHID), lambda i: (0, 0)),
                  pl.BlockSpec((DHID, 64), lambda i: (0, 0)),
                  pl.BlockSpec((1, 64), lambda i: (0, 0)),
                  pl.BlockSpec((64, nc), lambda i: (0, 0)),
                  pl.BlockSpec((1, nc), lambda i: (0, 0))],
        out_specs=pl.BlockSpec((_BN, nc), lambda i: (i, 0)),
        out_shape=jax.ShapeDtypeStruct((NN, nc), jnp.float32),
    )(acc, tab, dis, hn, bg, wf, bf, wc, bc)


def kernel(x, edge_index, h_node, W_gcn, b_gcn, W_fus, b_fus, W_cls, b_cls):
    e = edge_index.shape[1]
    epad = ((e + 4095) // 4096) * 4096
    npad = epad - e
    ar = jnp.arange(npad, dtype=jnp.int32)
    src = jnp.concatenate([edge_index[0], ar % NN])
    dst = jnp.concatenate([edge_index[1], NN + (ar % 16)])
    src2 = jnp.stack([src, src + NN])

    deg2 = _deg_sc(epad)(dst, jnp.zeros((TROW,), jnp.float32))
    xw = _xw_tc(x, W_gcn)
    tab, dis = _scale_tc(xw, deg2.reshape(NCORE, NP, 1))
    acc = _msg_sc(epad)(tab.reshape(NCORE * NN, HALF), src2, dst,
                        jnp.zeros((NP, HALF), jnp.float32))
    return _epi_tc(acc, tab, dis, h_node.reshape(NN, 1),
                   b_gcn.reshape(1, DHID), W_fus, b_fus.reshape(1, 64),
                   W_cls, b_cls.reshape(1, W_cls.shape[1]))


# speedup vs baseline: 19.1495x; 1.6473x over previous
import functools

import jax
import jax.numpy as jnp
from jax import lax
from jax.experimental import pallas as pl
from jax.experimental.pallas import tpu as pltpu
from jax.experimental.pallas import tpu_sc as plsc

NN = 10000
DIN = 256
DHID = 256
HALF = 128
NP = 10240
TROW = NP // 16
EB = 128
NCORE = 2
NSUB = 16

_mesh = plsc.VectorSubcoreMesh(core_axis_name="c", subcore_axis_name="s")


def _deg_sc(epad):
    nb = epad // (NCORE * NSUB * EB)

    @functools.partial(
        pl.kernel,
        mesh=_mesh,
        out_type=jax.ShapeDtypeStruct((NCORE, NP), jnp.float32),
        scratch_types=[
            pltpu.VMEM((nb, EB), jnp.int32),
            pltpu.VMEM((EB,), jnp.float32),
            pltpu.VMEM((TROW,), jnp.float32),
            pltpu.VMEM_SHARED((NP,), jnp.float32),
            pltpu.SemaphoreType.DMA,
        ],
    )
    def k(dst4_hbm, zrow_hbm, deg_out, didx_v, ones_v, zbuf_v, deg_sh, sem):
        c = lax.axis_index("c")
        s = lax.axis_index("s")
        wid = s * NCORE + c
        for i in range(EB // 16):
            ones_v[pl.ds(i * 16, 16)] = jnp.ones((16,), jnp.float32)
        pltpu.sync_copy(dst4_hbm.at[wid], didx_v)
        pltpu.sync_copy(zrow_hbm, zbuf_v)
        pltpu.sync_copy(zbuf_v, deg_sh.at[pl.ds(s * TROW, TROW)])
        plsc.subcore_barrier()

        def fire(b, carry):
            pltpu.async_copy(ones_v, deg_sh.at[didx_v.at[b]], sem, add=True)
            return carry

        lax.fori_loop(0, nb, fire, 0)

        def drain(b, carry):
            pltpu.make_async_copy(ones_v, deg_sh.at[didx_v.at[0]], sem).wait()
            return carry

        lax.fori_loop(0, nb, drain, 0)
        plsc.subcore_barrier()
        pltpu.sync_copy(deg_sh.at[pl.ds(s * TROW, TROW)],
                        deg_out.at[c, pl.ds(s * TROW, TROW)])

    return k


def _msg_sc(epad):
    nb = epad // (NSUB * EB)
    hb = nb // 2

    @functools.partial(
        pl.kernel,
        mesh=_mesh,
        out_type=jax.ShapeDtypeStruct((NCORE, NP, HALF), jnp.float32),
        scratch_types=[
            pltpu.VMEM((hb, EB), jnp.int32),
            pltpu.VMEM((hb, EB), jnp.int32),
            pltpu.VMEM((2, EB, HALF), jnp.float32),
            pltpu.VMEM_SHARED((NP, HALF), jnp.float32),
            pltpu.SemaphoreType.DMA((2,)),
        ],
    )
    def k(table_hbm, src4_hbm, dst3_hbm, zrows_hbm, acc_out,
          sidx_v, didx_v, rows_v, acc_sh, gsem):
        c = lax.axis_index("c")
        s = lax.axis_index("s")
        pltpu.sync_copy(zrows_hbm.at[pl.ds(s * TROW, TROW)],
                        acc_sh.at[pl.ds(s * TROW, TROW)])
        plsc.subcore_barrier()

        def body(i, carry):
            b = i * 2
            pltpu.async_copy(table_hbm.at[sidx_v.at[b + 1]], rows_v.at[1],
                             gsem.at[1])
            pltpu.make_async_copy(table_hbm.at[sidx_v.at[0]], rows_v.at[0],
                                  gsem.at[0]).wait()
            pltpu.sync_copy(rows_v.at[0], acc_sh.at[didx_v.at[b]], add=True)

            @pl.when(b + 2 < hb)
            def _():
                pltpu.async_copy(table_hbm.at[sidx_v.at[b + 2]], rows_v.at[0],
                                 gsem.at[0])

            pltpu.make_async_copy(table_hbm.at[sidx_v.at[0]], rows_v.at[1],
                                  gsem.at[1]).wait()
            pltpu.sync_copy(rows_v.at[1], acc_sh.at[didx_v.at[b + 1]],
                            add=True)
            return carry

        for p in range(2):
            pltpu.sync_copy(src4_hbm.at[c, s, pl.ds(p * hb, hb)], sidx_v)
            pltpu.sync_copy(dst3_hbm.at[s, pl.ds(p * hb, hb)], didx_v)
            pltpu.async_copy(table_hbm.at[sidx_v.at[0]], rows_v.at[0],
                             gsem.at[0])
            lax.fori_loop(0, hb // 2, body, 0)
        plsc.subcore_barrier()
        pltpu.sync_copy(acc_sh.at[pl.ds(s * TROW, TROW)],
                        acc_out.at[c, pl.ds(s * TROW, TROW)])

    return k


_BN = 400
_NBLK = NN // _BN


def _mm_body(x_ref, w_ref, o_ref):
    o_ref[...] = jnp.dot(x_ref[...], w_ref[...],
                         preferred_element_type=jnp.float32)


def _xw_tc(x, w):
    return pl.pallas_call(
        _mm_body,
        grid=(_NBLK,),
        in_specs=[pl.BlockSpec((_BN, DIN), lambda i: (i, 0)),
                  pl.BlockSpec((DIN, DHID), lambda i: (0, 0))],
        out_specs=pl.BlockSpec((_BN, DHID), lambda i: (i, 0)),
        out_shape=jax.ShapeDtypeStruct((NN, DHID), jnp.float32),
    )(x, w)


def _scale_body(xw_ref, deg_ref, tab_ref, dis_ref):
    deg = deg_ref[0] + deg_ref[1] + 1.0
    dis = lax.rsqrt(deg)
    dis_ref[...] = dis
    tab_ref[0] = xw_ref[:, :HALF] * dis
    tab_ref[1] = xw_ref[:, HALF:] * dis


def _scale_tc(xw, deg3):
    return pl.pallas_call(
        _scale_body,
        grid=(_NBLK,),
        in_specs=[pl.BlockSpec((_BN, DHID), lambda i: (i, 0)),
                  pl.BlockSpec((NCORE, _BN, 1), lambda i: (0, i, 0))],
        out_specs=[pl.BlockSpec((NCORE, _BN, HALF), lambda i: (0, i, 0)),
                   pl.BlockSpec((_BN, 1), lambda i: (i, 0))],
        out_shape=[jax.ShapeDtypeStruct((NCORE, NN, HALF), jnp.float32),
                   jax.ShapeDtypeStruct((NN, 1), jnp.float32)],
    )(xw, deg3)


def _epi_body(acc_ref, tab_ref, dis_ref, hn_ref, bg_ref, wf_ref, bf_ref,
              wc_ref, bc_ref, o_ref):
    a = jnp.concatenate(
        [acc_ref[0] + tab_ref[0], acc_ref[1] + tab_ref[1]], axis=1)
    h = jnp.maximum(a * dis_ref[...] + bg_ref[...], 0.0)
    ha = (1.0 - hn_ref[...]) * h
    h2 = jnp.maximum(
        jnp.dot(ha, wf_ref[...], preferred_element_type=jnp.float32)
        + bf_ref[...], 0.0)
    o_ref[...] = (jnp.dot(h2, wc_ref[...], preferred_element_type=jnp.float32)
                  + bc_ref[...])


def _epi_tc(acc, tab, dis, hn, bg, wf, bf, wc, bc):
    nc = wc.shape[1]
    return pl.pallas_call(
        _epi_body,
        grid=(_NBLK,),
        in_specs=[pl.BlockSpec((NCORE, _BN, HALF), lambda i: (0, i, 0)),
                  pl.BlockSpec((NCORE, _BN, HALF), lambda i: (0, i, 0)),
                  pl.BlockSpec((_BN, 1), lambda i: (i, 0)),
                  pl.BlockSpec((_BN, 1), lambda i: (i, 0)),
                  pl.BlockSpec((1, DHID), lambda i: (0, 0)),
                  pl.BlockSpec((DHID, 64), lambda i: (0, 0)),
                  pl.BlockSpec((1, 64), lambda i: (0, 0)),
                  pl.BlockSpec((64, nc), lambda i: (0, 0)),
                  pl.BlockSpec((1, nc), lambda i: (0, 0))],
        out_specs=pl.BlockSpec((_BN, nc), lambda i: (i, 0)),
        out_shape=jax.ShapeDtypeStruct((NN, nc), jnp.float32),
    )(acc, tab, dis, hn, bg, wf, bf, wc, bc)


def kernel(x, edge_index, h_node, W_gcn, b_gcn, W_fus, b_fus, W_cls, b_cls):
    e = edge_index.shape[1]
    epad = ((e + 4095) // 4096) * 4096
    npad = epad - e
    ar = jnp.arange(npad, dtype=jnp.int32)
    src = jnp.concatenate([edge_index[0], ar % NN])
    dst = jnp.concatenate([edge_index[1], NN + (ar % 16)])
    src2 = jnp.stack([src, src + NN])
    dst4 = dst.reshape(NCORE * NSUB, epad // (NCORE * NSUB * EB), EB)
    dst3 = dst.reshape(NSUB, epad // (NSUB * EB), EB)
    src4 = src2.reshape(NCORE, NSUB, epad // (NSUB * EB), EB)

    deg2 = _deg_sc(epad)(dst4, jnp.zeros((TROW,), jnp.float32))
    xw = _xw_tc(x, W_gcn)
    tab, dis = _scale_tc(xw, deg2.reshape(NCORE, NP, 1))
    acc = _msg_sc(epad)(tab.reshape(NCORE * NN, HALF), src4, dst3,
                        jnp.zeros((NP, HALF), jnp.float32))
    return _epi_tc(acc, tab, dis, h_node.reshape(NN, 1),
                   b_gcn.reshape(1, DHID), W_fus, b_fus.reshape(1, 64),
                   W_cls, b_cls.reshape(1, W_cls.shape[1]))


# speedup vs baseline: 22.3174x; 1.1654x over previous
import functools

import jax
import jax.numpy as jnp
from jax import lax
from jax.experimental import pallas as pl
from jax.experimental.pallas import tpu as pltpu
from jax.experimental.pallas import tpu_sc as plsc

NN = 10000
DIN = 256
DHID = 256
HALF = 128
NP = 10240
TROW = NP // 16
EB = 128
NCORE = 2
NSUB = 16

_mesh = plsc.VectorSubcoreMesh(core_axis_name="c", subcore_axis_name="s")


def _deg_sc(epad):
    nb = epad // (NCORE * NSUB * EB)

    @functools.partial(
        pl.kernel,
        mesh=_mesh,
        out_type=jax.ShapeDtypeStruct((NCORE, NP), jnp.float32),
        scratch_types=[
            pltpu.VMEM((nb, EB), jnp.int32),
            pltpu.VMEM((EB,), jnp.float32),
            pltpu.VMEM((TROW,), jnp.float32),
            pltpu.VMEM_SHARED((NP,), jnp.float32),
            pltpu.SemaphoreType.DMA,
        ],
    )
    def k(dst4_hbm, zrow_hbm, deg_out, didx_v, ones_v, zbuf_v, deg_sh, sem):
        c = lax.axis_index("c")
        s = lax.axis_index("s")
        wid = s * NCORE + c
        for i in range(EB // 16):
            ones_v[pl.ds(i * 16, 16)] = jnp.ones((16,), jnp.float32)
        pltpu.sync_copy(dst4_hbm.at[wid], didx_v)
        pltpu.sync_copy(zrow_hbm, zbuf_v)
        pltpu.sync_copy(zbuf_v, deg_sh.at[pl.ds(s * TROW, TROW)])
        plsc.subcore_barrier()

        def fire(b, carry):
            pltpu.async_copy(ones_v, deg_sh.at[didx_v.at[b]], sem, add=True)
            return carry

        lax.fori_loop(0, nb, fire, 0)

        def drain(b, carry):
            pltpu.make_async_copy(ones_v, deg_sh.at[didx_v.at[0]], sem).wait()
            return carry

        lax.fori_loop(0, nb, drain, 0)
        plsc.subcore_barrier()
        pltpu.sync_copy(deg_sh.at[pl.ds(s * TROW, TROW)],
                        deg_out.at[c, pl.ds(s * TROW, TROW)])

    return k


def _msg_sc(epad):
    nb = epad // (NSUB * EB)
    hb = nb // 2

    @functools.partial(
        pl.kernel,
        mesh=_mesh,
        out_type=jax.ShapeDtypeStruct((NCORE, NP, HALF), jnp.float32),
        scratch_types=[
            pltpu.VMEM((hb, EB), jnp.int32),
            pltpu.VMEM((hb, EB), jnp.int32),
            pltpu.VMEM((2, EB, HALF), jnp.float32),
            pltpu.VMEM_SHARED((NP, HALF), jnp.float32),
            pltpu.SemaphoreType.DMA((2,)),
        ],
    )
    def k(table_hbm, src4_hbm, dst3_hbm, zrows_hbm, acc_out,
          sidx_v, didx_v, rows_v, acc_sh, gsem):
        c = lax.axis_index("c")
        s = lax.axis_index("s")
        pltpu.sync_copy(zrows_hbm.at[pl.ds(s * TROW, TROW)],
                        acc_sh.at[pl.ds(s * TROW, TROW)])
        plsc.subcore_barrier()

        def body(i, carry):
            b = i * 2
            pltpu.async_copy(table_hbm.at[sidx_v.at[b + 1]], rows_v.at[1],
                             gsem.at[1])
            pltpu.make_async_copy(table_hbm.at[sidx_v.at[0]], rows_v.at[0],
                                  gsem.at[0]).wait()
            pltpu.sync_copy(rows_v.at[0], acc_sh.at[didx_v.at[b]], add=True)

            @pl.when(b + 2 < hb)
            def _():
                pltpu.async_copy(table_hbm.at[sidx_v.at[b + 2]], rows_v.at[0],
                                 gsem.at[0])

            pltpu.make_async_copy(table_hbm.at[sidx_v.at[0]], rows_v.at[1],
                                  gsem.at[1]).wait()
            pltpu.sync_copy(rows_v.at[1], acc_sh.at[didx_v.at[b + 1]],
                            add=True)
            return carry

        for p in range(2):
            pltpu.sync_copy(src4_hbm.at[c, s, pl.ds(p * hb, hb)], sidx_v)
            pltpu.sync_copy(dst3_hbm.at[s, pl.ds(p * hb, hb)], didx_v)
            pltpu.async_copy(table_hbm.at[sidx_v.at[0]], rows_v.at[0],
                             gsem.at[0])
            lax.fori_loop(0, hb // 2, body, 0)
        plsc.subcore_barrier()
        pltpu.sync_copy(acc_sh.at[pl.ds(s * TROW, TROW)],
                        acc_out.at[c, pl.ds(s * TROW, TROW)])

    return k


_BN = 1000
_NBLK = NN // _BN


def _xws_body(x_ref, w_ref, deg_ref, tab_ref, dis_ref):
    xw = jnp.dot(x_ref[...], w_ref[...], preferred_element_type=jnp.float32)
    deg = deg_ref[0] + deg_ref[1] + 1.0
    dis = lax.rsqrt(deg)
    dis_ref[...] = dis
    tab_ref[0] = xw[:, :HALF] * dis
    tab_ref[1] = xw[:, HALF:] * dis


def _xws_tc(x, w, deg3):
    return pl.pallas_call(
        _xws_body,
        grid=(_NBLK,),
        in_specs=[pl.BlockSpec((_BN, DIN), lambda i: (i, 0)),
                  pl.BlockSpec((DIN, DHID), lambda i: (0, 0)),
                  pl.BlockSpec((NCORE, _BN, 1), lambda i: (0, i, 0))],
        out_specs=[pl.BlockSpec((NCORE, _BN, HALF), lambda i: (0, i, 0)),
                   pl.BlockSpec((_BN, 1), lambda i: (i, 0))],
        out_shape=[jax.ShapeDtypeStruct((NCORE, NN, HALF), jnp.float32),
                   jax.ShapeDtypeStruct((NN, 1), jnp.float32)],
    )(x, w, deg3)


def _epi_body(acc_ref, tab_ref, dis_ref, hn_ref, bg_ref, wf_ref, bf_ref,
              wc_ref, bc_ref, o_ref):
    a = jnp.concatenate(
        [acc_ref[0] + tab_ref[0], acc_ref[1] + tab_ref[1]], axis=1)
    h = jnp.maximum(a * dis_ref[...] + bg_ref[...], 0.0)
    ha = (1.0 - hn_ref[...]) * h
    h2 = jnp.maximum(
        jnp.dot(ha, wf_ref[...], preferred_element_type=jnp.float32)
        + bf_ref[...], 0.0)
    o_ref[...] = (jnp.dot(h2, wc_ref[...], preferred_element_type=jnp.float32)
                  + bc_ref[...])


def _epi_tc(acc, tab, dis, hn, bg, wf, bf, wc, bc):
    nc = wc.shape[1]
    return pl.pallas_call(
        _epi_body,
        grid=(_NBLK,),
        in_specs=[pl.BlockSpec((NCORE, _BN, HALF), lambda i: (0, i, 0)),
                  pl.BlockSpec((NCORE, _BN, HALF), lambda i: (0, i, 0)),
                  pl.BlockSpec((_BN, 1), lambda i: (i, 0)),
                  pl.BlockSpec((_BN, 1), lambda i: (i, 0)),
                  pl.BlockSpec((1, DHID), lambda i: (0, 0)),
                  pl.BlockSpec((DHID, 64), lambda i: (0, 0)),
                  pl.BlockSpec((1, 64), lambda i: (0, 0)),
                  pl.BlockSpec((64, nc), lambda i: (0, 0)),
                  pl.BlockSpec((1, nc), lambda i: (0, 0))],
        out_specs=pl.BlockSpec((_BN, nc), lambda i: (i, 0)),
        out_shape=jax.ShapeDtypeStruct((NN, nc), jnp.float32),
    )(acc, tab, dis, hn, bg, wf, bf, wc, bc)


def kernel(x, edge_index, h_node, W_gcn, b_gcn, W_fus, b_fus, W_cls, b_cls):
    e = edge_index.shape[1]
    epad = ((e + 4095) // 4096) * 4096
    npad = epad - e
    ar = jnp.arange(npad, dtype=jnp.int32)
    src = jnp.concatenate([edge_index[0], ar % NN])
    dst = jnp.concatenate([edge_index[1], NN + (ar % 16)])
    src2 = jnp.stack([src, src + NN])
    dst4 = dst.reshape(NCORE * NSUB, epad // (NCORE * NSUB * EB), EB)
    dst3 = dst.reshape(NSUB, epad // (NSUB * EB), EB)
    src4 = src2.reshape(NCORE, NSUB, epad // (NSUB * EB), EB)

    deg2 = _deg_sc(epad)(dst4, jnp.zeros((TROW,), jnp.float32))
    tab, dis = _xws_tc(x.astype(jnp.bfloat16), W_gcn.astype(jnp.bfloat16),
                       deg2.reshape(NCORE, NP, 1))
    acc = _msg_sc(epad)(tab.reshape(NCORE * NN, HALF), src4, dst3,
                        jnp.zeros((NP, HALF), jnp.float32))
    return _epi_tc(acc, tab, dis, h_node.reshape(NN, 1),
                   b_gcn.reshape(1, DHID), W_fus, b_fus.reshape(1, 64),
                   W_cls, b_cls.reshape(1, W_cls.shape[1]))


# speedup vs baseline: 23.6551x; 1.0599x over previous
import functools

import jax
import jax.numpy as jnp
from jax import lax
from jax.experimental import pallas as pl
from jax.experimental.pallas import tpu as pltpu
from jax.experimental.pallas import tpu_sc as plsc

NN = 10000
DIN = 256
DHID = 256
HALF = 128
NP = 10240
TROW = NP // 16
EE = 160000
EB = 80
NB = EE // (16 * EB)
NCORE = 2
NSUB = 16

_mesh = plsc.VectorSubcoreMesh(core_axis_name="c", subcore_axis_name="s")


@functools.partial(
    pl.kernel,
    mesh=_mesh,
    out_type=jax.ShapeDtypeStruct((NCORE, NP), jnp.float32),
    scratch_types=[
        pltpu.VMEM((NB, EB), jnp.int32),
        pltpu.VMEM((EB,), jnp.float32),
        pltpu.VMEM((TROW,), jnp.float32),
        pltpu.VMEM_SHARED((NP,), jnp.float32),
        pltpu.SemaphoreType.DMA,
    ],
)
def _deg_sc(e4_hbm, zrow_hbm, deg_out, didx_v, ones_v, zbuf_v, deg_sh, sem):
    c = lax.axis_index("c")
    s = lax.axis_index("s")
    for i in range(EB // 16):
        ones_v[pl.ds(i * 16, 16)] = jnp.ones((16,), jnp.float32)
    pltpu.sync_copy(e4_hbm.at[1, s], didx_v)
    pltpu.sync_copy(zrow_hbm, zbuf_v)
    pltpu.sync_copy(zbuf_v, deg_sh.at[pl.ds(s * TROW, TROW)])
    plsc.subcore_barrier()

    nmine = (NB + 1) // 2 - c

    def fire(b, carry):
        pltpu.async_copy(ones_v, deg_sh.at[didx_v.at[2 * b + c]], sem,
                         add=True)
        return carry

    lax.fori_loop(0, nmine, fire, 0)

    def drain(b, carry):
        pltpu.make_async_copy(ones_v, deg_sh.at[didx_v.at[0]], sem).wait()
        return carry

    lax.fori_loop(0, nmine, drain, 0)
    plsc.subcore_barrier()
    pltpu.sync_copy(deg_sh.at[pl.ds(s * TROW, TROW)],
                    deg_out.at[c, pl.ds(s * TROW, TROW)])


@functools.partial(
    pl.kernel,
    mesh=_mesh,
    out_type=jax.ShapeDtypeStruct((NCORE, NP, HALF), jnp.float32),
    scratch_types=[
        pltpu.VMEM((NB * EB,), jnp.int32),
        pltpu.VMEM((NB, EB), jnp.int32),
        pltpu.VMEM((2, EB, HALF), jnp.float32),
        pltpu.VMEM_SHARED((NP, HALF), jnp.float32),
        pltpu.SemaphoreType.DMA((2,)),
    ],
)
def _msg_sc(tab3_hbm, e3_hbm, e4_hbm, zrows_hbm, acc_out,
            sidx_v, didx_v, rows_v, acc_sh, gsem):
    c = lax.axis_index("c")
    s = lax.axis_index("s")
    pltpu.sync_copy(e3_hbm.at[0, s], sidx_v)
    pltpu.sync_copy(e4_hbm.at[1, s], didx_v)
    pltpu.sync_copy(zrows_hbm.at[pl.ds(s * TROW, TROW)],
                    acc_sh.at[pl.ds(s * TROW, TROW)])
    plsc.subcore_barrier()

    pltpu.async_copy(tab3_hbm.at[c].at[sidx_v.at[pl.ds(0, EB)]],
                     rows_v.at[0], gsem.at[0])

    def body(b, carry):
        slot = lax.rem(b, 2)
        nslot = lax.rem(b + 1, 2)

        @pl.when(b + 1 < NB)
        def _():
            pltpu.async_copy(
                tab3_hbm.at[c].at[sidx_v.at[pl.ds((b + 1) * EB, EB)]],
                rows_v.at[nslot], gsem.at[nslot])

        pltpu.make_async_copy(tab3_hbm.at[c].at[sidx_v.at[pl.ds(0, EB)]],
                              rows_v.at[slot], gsem.at[slot]).wait()
        pltpu.sync_copy(rows_v.at[slot], acc_sh.at[didx_v.at[b]], add=True)
        return carry

    lax.fori_loop(0, NB, body, 0)
    plsc.subcore_barrier()
    pltpu.sync_copy(acc_sh.at[pl.ds(s * TROW, TROW)],
                    acc_out.at[c, pl.ds(s * TROW, TROW)])


_BN = 1024
_NBLK = NP // _BN


def _xws_body(x_ref, w_ref, deg_ref, tab_ref, dis_ref):
    xw = jnp.dot(x_ref[...].astype(jnp.bfloat16),
                 w_ref[...].astype(jnp.bfloat16),
                 preferred_element_type=jnp.float32)
    dt = jnp.transpose(deg_ref[...], (1, 0))
    dis = lax.rsqrt(dt[:, 0:1] + dt[:, 1:2] + 1.0)
    dis_ref[...] = dis
    tab_ref[0] = xw[:, :HALF] * dis
    tab_ref[1] = xw[:, HALF:] * dis


def _xws_tc(x, w, deg2):
    return pl.pallas_call(
        _xws_body,
        grid=(_NBLK,),
        in_specs=[pl.BlockSpec((_BN, DIN), lambda i: (i, 0)),
                  pl.BlockSpec((DIN, DHID), lambda i: (0, 0)),
                  pl.BlockSpec((NCORE, _BN), lambda i: (0, i))],
        out_specs=[pl.BlockSpec((NCORE, _BN, HALF), lambda i: (0, i, 0)),
                   pl.BlockSpec((_BN, 1), lambda i: (i, 0))],
        out_shape=[jax.ShapeDtypeStruct((NCORE, NP, HALF), jnp.float32),
                   jax.ShapeDtypeStruct((NP, 1), jnp.float32)],
    )(x, w, deg2)


def _epi_body(acc_ref, tab_ref, dis_ref, hn_ref, bg_ref, wf_ref, bf_ref,
              wc_ref, bc_ref, o_ref):
    dis = dis_ref[...]
    a = jnp.concatenate(
        [(acc_ref[0] + tab_ref[0]) * dis, (acc_ref[1] + tab_ref[1]) * dis],
        axis=1)
    h = jnp.maximum(a + bg_ref[...], 0.0)
    alpha = 1.0 - jnp.transpose(hn_ref[...], (1, 0))
    ha = alpha * h
    h2 = jnp.maximum(
        jnp.dot(ha, wf_ref[...], preferred_element_type=jnp.float32)
        + bf_ref[...], 0.0)
    out = (jnp.dot(h2, wc_ref[...], preferred_element_type=jnp.float32)
           + bc_ref[...])
    o_ref[...] = jnp.transpose(out, (1, 0))


def _epi_tc(acc, tab, dis, hn, bg, wf, bf, wc, bc):
    nc = wc.shape[1]
    return pl.pallas_call(
        _epi_body,
        grid=(_NBLK,),
        in_specs=[pl.BlockSpec((NCORE, _BN, HALF), lambda i: (0, i, 0)),
                  pl.BlockSpec((NCORE, _BN, HALF), lambda i: (0, i, 0)),
                  pl.BlockSpec((_BN, 1), lambda i: (i, 0)),
                  pl.BlockSpec((1, _BN), lambda i: (0, i)),
                  pl.BlockSpec((1, DHID), lambda i: (0, 0)),
                  pl.BlockSpec((DHID, 64), lambda i: (0, 0)),
                  pl.BlockSpec((1, 64), lambda i: (0, 0)),
                  pl.BlockSpec((64, nc), lambda i: (0, 0)),
                  pl.BlockSpec((1, nc), lambda i: (0, 0))],
        out_specs=pl.BlockSpec((nc, _BN), lambda i: (0, i)),
        out_shape=jax.ShapeDtypeStruct((nc, NN), jnp.float32),
    )(acc, tab, dis, hn, bg, wf, bf, wc, bc)


def kernel(x, edge_index, h_node, W_gcn, b_gcn, W_fus, b_fus, W_cls, b_cls):
    e4 = edge_index.reshape(2, NSUB, NB, EB)
    e3 = edge_index.reshape(2, NSUB, NB * EB)
    deg2 = _deg_sc(e4, jnp.zeros((TROW,), jnp.float32))
    tab, dis = _xws_tc(x, W_gcn, deg2)
    acc = _msg_sc(tab, e3, e4, jnp.zeros((NP, HALF), jnp.float32))
    out_t = _epi_tc(acc, tab, dis, h_node.reshape(1, NN),
                    b_gcn.reshape(1, DHID), W_fus, b_fus.reshape(1, 64),
                    W_cls, b_cls.reshape(1, W_cls.shape[1]))
    return out_t.T


# speedup vs baseline: 25.1764x; 1.0643x over previous
import functools

import jax
import jax.numpy as jnp
from jax import lax
from jax.experimental import pallas as pl
from jax.experimental.pallas import tpu as pltpu
from jax.experimental.pallas import tpu_sc as plsc

NN = 10000
DIN = 256
DHID = 256
HALF = 128
NP = 10240
TROW = NP // 16
EB = 128
ET = 10000
NBT = 79
ETP = NBT * EB
PADE = ETP - ET
NCORE = 2
NSUB = 16
PH = (40, 39)

_mesh = plsc.VectorSubcoreMesh(core_axis_name="c", subcore_axis_name="s")


@functools.partial(
    pl.kernel,
    mesh=_mesh,
    out_type=jax.ShapeDtypeStruct((NCORE, NP), jnp.float32),
    scratch_types=[
        pltpu.VMEM((NBT, EB), jnp.int32),
        pltpu.VMEM((EB,), jnp.float32),
        pltpu.VMEM((TROW,), jnp.float32),
        pltpu.VMEM_SHARED((NP,), jnp.float32),
        pltpu.SemaphoreType.DMA,
    ],
)
def _deg_sc(dst5_hbm, deg_out, didx_v, ones_v, zbuf_v, deg_sh, sem):
    c = lax.axis_index("c")
    s = lax.axis_index("s")
    for i in range(EB // 16):
        ones_v[pl.ds(i * 16, 16)] = jnp.ones((16,), jnp.float32)
    for i in range(TROW // 16):
        zbuf_v[pl.ds(i * 16, 16)] = jnp.zeros((16,), jnp.float32)
    pltpu.sync_copy(dst5_hbm.at[s], didx_v)
    pltpu.sync_copy(zbuf_v, deg_sh.at[pl.ds(s * TROW, TROW)])
    plsc.subcore_barrier()

    nmine = (NBT + 1) // 2 - c

    def fire(b, carry):
        pltpu.async_copy(ones_v, deg_sh.at[didx_v.at[2 * b + c]], sem,
                         add=True)
        return carry

    lax.fori_loop(0, nmine, fire, 0)

    def drain(b, carry):
        pltpu.make_async_copy(ones_v, deg_sh.at[didx_v.at[0]], sem).wait()
        return carry

    lax.fori_loop(0, nmine, drain, 0)
    plsc.subcore_barrier()
    pltpu.sync_copy(deg_sh.at[pl.ds(s * TROW, TROW)],
                    deg_out.at[c, pl.ds(s * TROW, TROW)])


@functools.partial(
    pl.kernel,
    mesh=_mesh,
    out_type=jax.ShapeDtypeStruct((NCORE, NP, HALF), jnp.float32),
    scratch_types=[
        pltpu.VMEM((PH[0] * EB,), jnp.int32),
        pltpu.VMEM((PH[0], EB), jnp.int32),
        pltpu.VMEM((2, EB, HALF), jnp.float32),
        pltpu.VMEM_SHARED((NP, HALF), jnp.float32),
        pltpu.SemaphoreType.DMA((2,)),
    ],
)
def _msg_sc(tab3_hbm, src5_hbm, dst5_hbm, acc_out,
            sidx_v, didx_v, rows_v, acc_sh, gsem):
    c = lax.axis_index("c")
    s = lax.axis_index("s")
    def zfill(r, carry):
        for j in range(HALF // 16):
            rows_v[0, r, pl.ds(j * 16, 16)] = jnp.zeros((16,), jnp.float32)
        return carry

    lax.fori_loop(0, EB, zfill, 0)
    for k in range(TROW // EB):
        pltpu.sync_copy(rows_v.at[0],
                        acc_sh.at[pl.ds(s * TROW + k * EB, EB)])
    plsc.subcore_barrier()

    def body(b, carry):
        slot = lax.rem(b, 2)
        nslot = lax.rem(b + 1, 2)
        nbp = carry

        @pl.when(b + 1 < nbp)
        def _():
            pltpu.async_copy(
                tab3_hbm.at[c].at[sidx_v.at[pl.ds((b + 1) * EB, EB)]],
                rows_v.at[nslot], gsem.at[nslot])

        pltpu.make_async_copy(tab3_hbm.at[c].at[sidx_v.at[pl.ds(0, EB)]],
                              rows_v.at[slot], gsem.at[slot]).wait()
        pltpu.sync_copy(rows_v.at[slot], acc_sh.at[didx_v.at[b]], add=True)
        return carry

    base = 0
    for nbp in PH:
        pltpu.sync_copy(src5_hbm.at[s, pl.ds(base * EB, nbp * EB)],
                        sidx_v.at[pl.ds(0, nbp * EB)])
        pltpu.sync_copy(dst5_hbm.at[s, pl.ds(base, nbp)],
                        didx_v.at[pl.ds(0, nbp)])
        pltpu.async_copy(tab3_hbm.at[c].at[sidx_v.at[pl.ds(0, EB)]],
                         rows_v.at[0], gsem.at[0])
        lax.fori_loop(0, nbp, body, nbp)
        base += nbp
    plsc.subcore_barrier()
    pltpu.sync_copy(acc_sh.at[pl.ds(s * TROW, TROW)],
                    acc_out.at[c, pl.ds(s * TROW, TROW)])


_BN = 1024
_NBLK = NP // _BN


def _xws_body(x_ref, w_ref, deg_ref, tab_ref, dis_ref):
    xw = jnp.dot(x_ref[...].astype(jnp.bfloat16),
                 w_ref[...].astype(jnp.bfloat16),
                 preferred_element_type=jnp.float32)
    dt = jnp.transpose(deg_ref[...], (1, 0))
    dis = lax.rsqrt(dt[:, 0:1] + dt[:, 1:2] + 1.0)
    dis_ref[...] = dis
    tab_ref[0] = xw[:, :HALF] * dis
    tab_ref[1] = xw[:, HALF:] * dis


def _xws_tc(x, w, deg2):
    return pl.pallas_call(
        _xws_body,
        grid=(_NBLK,),
        in_specs=[pl.BlockSpec((_BN, DIN), lambda i: (i, 0)),
                  pl.BlockSpec((DIN, DHID), lambda i: (0, 0)),
                  pl.BlockSpec((NCORE, _BN), lambda i: (0, i))],
        out_specs=[pl.BlockSpec((NCORE, _BN, HALF), lambda i: (0, i, 0)),
                   pl.BlockSpec((_BN, 1), lambda i: (i, 0))],
        out_shape=[jax.ShapeDtypeStruct((NCORE, NP, HALF), jnp.float32),
                   jax.ShapeDtypeStruct((NP, 1), jnp.float32)],
    )(x, w, deg2)


def _epi_body(acc_ref, tab_ref, dis_ref, hn_ref, bg_ref, wf_ref, bf_ref,
              wc_ref, bc_ref, o_ref):
    dis = dis_ref[...]
    a = jnp.concatenate(
        [(acc_ref[0] + tab_ref[0]) * dis, (acc_ref[1] + tab_ref[1]) * dis],
        axis=1)
    h = jnp.maximum(a + bg_ref[...], 0.0)
    alpha = 1.0 - jnp.transpose(hn_ref[...], (1, 0))
    ha = (alpha * h).astype(jnp.bfloat16)
    h2 = jnp.maximum(
        jnp.dot(ha, wf_ref[...].astype(jnp.bfloat16),
                preferred_element_type=jnp.float32) + bf_ref[...], 0.0)
    out = (jnp.dot(h2, wc_ref[...], preferred_element_type=jnp.float32)
           + bc_ref[...])
    o_ref[...] = jnp.transpose(out, (1, 0))


def _epi_tc(acc, tab, dis, hn, bg, wf, bf, wc, bc):
    nc = wc.shape[1]
    return pl.pallas_call(
        _epi_body,
        grid=(_NBLK,),
        in_specs=[pl.BlockSpec((NCORE, _BN, HALF), lambda i: (0, i, 0)),
                  pl.BlockSpec((NCORE, _BN, HALF), lambda i: (0, i, 0)),
                  pl.BlockSpec((_BN, 1), lambda i: (i, 0)),
                  pl.BlockSpec((1, _BN), lambda i: (0, i)),
                  pl.BlockSpec((1, DHID), lambda i: (0, 0)),
                  pl.BlockSpec((DHID, 64), lambda i: (0, 0)),
                  pl.BlockSpec((1, 64), lambda i: (0, 0)),
                  pl.BlockSpec((64, nc), lambda i: (0, 0)),
                  pl.BlockSpec((1, nc), lambda i: (0, 0))],
        out_specs=pl.BlockSpec((nc, _BN), lambda i: (0, i)),
        out_shape=jax.ShapeDtypeStruct((nc, NN), jnp.float32),
    )(acc, tab, dis, hn, bg, wf, bf, wc, bc)


def kernel(x, edge_index, h_node, W_gcn, b_gcn, W_fus, b_fus, W_cls, b_cls):
    ar = jnp.arange(PADE, dtype=jnp.int32)
    tr = jnp.arange(NSUB, dtype=jnp.int32)
    pad_src = (tr[:, None] * 7919 + ar[None, :] * 89) % NN
    pad_dst = jnp.broadcast_to(NN + (ar[None, :] % 16), (NSUB, PADE))
    src5 = jnp.concatenate([edge_index[0].reshape(NSUB, ET), pad_src], axis=1)
    dst5 = jnp.concatenate([edge_index[1].reshape(NSUB, ET), pad_dst],
                           axis=1).reshape(NSUB, NBT, EB)

    deg2 = _deg_sc(dst5)
    tab, dis = _xws_tc(x, W_gcn, deg2)
    acc = _msg_sc(tab, src5, dst5)
    out_t = _epi_tc(acc, tab, dis, h_node.reshape(1, NN),
                    b_gcn.reshape(1, DHID), W_fus, b_fus.reshape(1, 64),
                    W_cls, b_cls.reshape(1, W_cls.shape[1]))
    return out_t.T
